# Initial kernel scaffold; baseline (speedup 1.0000x reference)
import jax
import jax.numpy as jnp
from jax.experimental import pallas as pl


def kernel(positions, senders, receivers):
    raise NotImplementedError("write your pallas kernel here")


# speedup vs baseline: 21.6912x; 21.6912x over previous
import functools

import jax
import jax.numpy as jnp
from jax import lax
from jax.experimental import pallas as pl
from jax.experimental.pallas import tpu as pltpu
from jax.experimental.pallas import tpu_sc as plsc

N_NODES = 100000
N_PAD = 102400
N_EDGES = 6400000
CHUNK = 512
CHUNKS = 391
NW = 32
E_PAD = NW * CHUNKS * CHUNK
MERGE_ROUNDS = 10
MERGE_RN = N_PAD // MERGE_ROUNDS
MERGE_WIN = MERGE_RN // 16

_mesh = plsc.VectorSubcoreMesh(core_axis_name="c", subcore_axis_name="s")


@functools.partial(
    pl.kernel,
    mesh=_mesh,
    compiler_params=pltpu.CompilerParams(needs_layout_passes=False),
    out_type=jax.ShapeDtypeStruct((2, N_PAD), jnp.float32),
    scratch_types=[
        pltpu.VMEM((CHUNK,), jnp.int32),
        pltpu.VMEM((CHUNK,), jnp.int32),
        pltpu.VMEM((3 * CHUNK,), jnp.int32),
        pltpu.VMEM((3 * CHUNK,), jnp.int32),
        pltpu.VMEM((3 * CHUNK,), jnp.float32),
        pltpu.VMEM((3 * CHUNK,), jnp.float32),
        pltpu.VMEM((N_PAD,), jnp.float32),
        pltpu.VMEM((MERGE_WIN,), jnp.float32),
        pltpu.VMEM((MERGE_WIN,), jnp.float32),
        pltpu.VMEM_SHARED((16 * MERGE_RN,), jnp.float32),
        pltpu.SemaphoreType.DMA,
    ],
)
def _lj_sc(posf_hbm, s_hbm, r_hbm, out_hbm,
           sidx_v, ridx_v, sfid_v, rfid_v, scoord_v, rcoord_v,
           accum_v, accs_v, tmp_v, shared_v, sem):
    cid = lax.axis_index("c")
    sid = lax.axis_index("s")
    wid = sid * 2 + cid
    wbase = wid * (CHUNKS * CHUNK)

    zero16 = jnp.zeros((16,), jnp.float32)
    lane = lax.iota(jnp.int32, 16)
    lane3 = lane * 3

    def zero_body(i, _):
        accum_v[pl.ds(i * 16, 16)] = zero16
        return _
    lax.fori_loop(0, N_PAD // 16, zero_body, None)

    def expand16(j, _):
        s16 = sidx_v[pl.ds(j * 16, 16)]
        r16 = ridx_v[pl.ds(j * 16, 16)]
        sf = s16 * 3
        rf = r16 * 3
        tgt = j * 48 + lane3
        plsc.store_scatter(sfid_v, [tgt], sf)
        plsc.store_scatter(sfid_v, [tgt + 1], sf + 1)
        plsc.store_scatter(sfid_v, [tgt + 2], sf + 2)
        plsc.store_scatter(rfid_v, [tgt], rf)
        plsc.store_scatter(rfid_v, [tgt + 1], rf + 1)
        plsc.store_scatter(rfid_v, [tgt + 2], rf + 2)
        return _

    def edge16(j, _):
        base3 = j * 48
        sx = plsc.load_gather(scoord_v, [base3 + lane3])
        sy = plsc.load_gather(scoord_v, [base3 + lane3 + 1])
        sz = plsc.load_gather(scoord_v, [base3 + lane3 + 2])
        rx = plsc.load_gather(rcoord_v, [base3 + lane3])
        ry = plsc.load_gather(rcoord_v, [base3 + lane3 + 1])
        rz = plsc.load_gather(rcoord_v, [base3 + lane3 + 2])
        dx = rx - sx
        dy = ry - sy
        dz = rz - sz
        r2 = dx * dx + dy * dy + dz * dz
        inv = 1.0 / r2
        e = inv * inv * inv
        en = 2.0 * (e * e - e)
        ridx16 = ridx_v[pl.ds(j * 16, 16)]
        plsc.addupdate_scatter(accum_v, [ridx16], en)
        return _

    def chunk_body(t, _):
        base = wbase + t * CHUNK
        pltpu.sync_copy(s_hbm.at[pl.ds(base, CHUNK)], sidx_v)
        pltpu.sync_copy(r_hbm.at[pl.ds(base, CHUNK)], ridx_v)
        lax.fori_loop(0, CHUNK // 16, expand16, None)
        copies = []
        for k in range(3 * CHUNK // 128):
            sl = pl.ds(k * 128, 128)
            copies.append(pltpu.async_copy(
                posf_hbm.at[sfid_v.at[sl]], scoord_v.at[sl], sem))
            copies.append(pltpu.async_copy(
                posf_hbm.at[rfid_v.at[sl]], rcoord_v.at[sl], sem))
        for cp in copies:
            cp.wait()
        lax.fori_loop(0, CHUNK // 16, edge16, None)
        return _

    lax.fori_loop(0, CHUNKS, chunk_body, None)

    def add_body(i, _):
        plsc.addupdate(accs_v.at[pl.ds(i * 16, 16)], tmp_v[pl.ds(i * 16, 16)])
        return _

    off = sid * MERGE_WIN
    for rr in range(MERGE_ROUNDS):
        pltpu.sync_copy(accum_v.at[pl.ds(rr * MERGE_RN, MERGE_RN)],
                        shared_v.at[pl.ds(sid * MERGE_RN, MERGE_RN)])
        plsc.subcore_barrier()
        pltpu.sync_copy(shared_v.at[pl.ds(off, MERGE_WIN)], accs_v)
        for src in range(1, 16):
            pltpu.sync_copy(
                shared_v.at[pl.ds(src * MERGE_RN + off, MERGE_WIN)], tmp_v)
            lax.fori_loop(0, MERGE_WIN // 16, add_body, None)
        pltpu.sync_copy(accs_v, out_hbm.at[cid, pl.ds(rr * MERGE_RN + off, MERGE_WIN)])
        plsc.subcore_barrier()


def _tc_sum_body(a_ref, o_ref):
    o_ref[...] = a_ref[0] + a_ref[1]


_tc_sum = pl.pallas_call(
    _tc_sum_body,
    out_shape=jax.ShapeDtypeStruct((N_PAD // 128, 128), jnp.float32),
)


def kernel(positions, senders, receivers):
    pos_flat = jnp.concatenate(
        [positions.reshape(-1), jnp.zeros((3 * (N_PAD - N_NODES),), jnp.float32)])
    pad = E_PAD - N_EDGES
    s_pad = jnp.concatenate([senders, jnp.zeros((pad,), jnp.int32)])
    r_pad = jnp.concatenate([receivers, jnp.full((pad,), N_NODES, jnp.int32)])
    partials = _lj_sc(pos_flat, s_pad, r_pad)
    summed = _tc_sum(partials.reshape(2, N_PAD // 128, 128))
    return summed.reshape(-1)[:N_NODES]


# speedup vs baseline: 25.1636x; 1.1601x over previous
import functools

import jax
import jax.numpy as jnp
from jax import lax
from jax.experimental import pallas as pl
from jax.experimental.pallas import tpu as pltpu
from jax.experimental.pallas import tpu_sc as plsc

N_NODES = 100000
N_PAD = 102400
N_EDGES = 6400000
CHUNK = 256
CHUNKS = 784
NW = 32
E_PAD = NW * CHUNKS * CHUNK
MERGE_ROUNDS = 10
MERGE_RN = N_PAD // MERGE_ROUNDS
MERGE_WIN = MERGE_RN // 16

_mesh = plsc.VectorSubcoreMesh(core_axis_name="c", subcore_axis_name="s")


@functools.partial(
    pl.kernel,
    mesh=_mesh,
    compiler_params=pltpu.CompilerParams(needs_layout_passes=False),
    out_type=jax.ShapeDtypeStruct((2, N_PAD), jnp.float32),
    scratch_types=(
        [pltpu.VMEM((CHUNK,), jnp.int32)] * 4 +
        [pltpu.VMEM((CHUNK,), jnp.int32)] * 4 +
        [pltpu.VMEM((3 * CHUNK,), jnp.int32)] * 2 +
        [pltpu.VMEM((3 * CHUNK,), jnp.int32)] * 2 +
        [pltpu.VMEM((3 * CHUNK,), jnp.float32)] * 2 +
        [pltpu.VMEM((3 * CHUNK,), jnp.float32)] * 2 +
        [pltpu.VMEM((N_PAD,), jnp.float32),
         pltpu.VMEM((MERGE_WIN,), jnp.float32),
         pltpu.VMEM((MERGE_WIN,), jnp.float32),
         pltpu.VMEM_SHARED((16 * MERGE_RN,), jnp.float32),
         pltpu.SemaphoreType.DMA,
         pltpu.SemaphoreType.DMA]
    ),
)
def _lj_sc(posf_hbm, s_hbm, r_hbm, out_hbm, *scr):
    SIDX = scr[0:4]
    RIDX = scr[4:8]
    SFID = scr[8:10]
    RFID = scr[10:12]
    SCO = scr[12:14]
    RCO = scr[14:16]
    accum_v, accs_v, tmp_v, shared_v, isem, gsem = scr[16:22]
    cid = lax.axis_index("c")
    sid = lax.axis_index("s")
    wid = sid * 2 + cid
    wbase = wid * (CHUNKS * CHUNK)

    zero16 = jnp.zeros((16,), jnp.float32)
    lane = lax.iota(jnp.int32, 16)
    lane3 = lane * 3

    def zero_body(i, _):
        accum_v[pl.ds(i * 16, 16)] = zero16
        return _
    lax.fori_loop(0, N_PAD // 16, zero_body, None)

    def start_idx(t, s):
        base = wbase + t * CHUNK
        pltpu.async_copy(s_hbm.at[pl.ds(base, CHUNK)], SIDX[s], isem)
        pltpu.async_copy(r_hbm.at[pl.ds(base, CHUNK)], RIDX[s], isem)

    def wait_idx(s):
        pltpu.make_async_copy(s_hbm.at[pl.ds(0, CHUNK)], SIDX[s], isem).wait()
        pltpu.make_async_copy(r_hbm.at[pl.ds(0, CHUNK)], RIDX[s], isem).wait()

    def expand(s, b):
        def expand16(j, _):
            s16 = SIDX[s][pl.ds(j * 16, 16)]
            r16 = RIDX[s][pl.ds(j * 16, 16)]
            sf = s16 * 3
            rf = r16 * 3
            tgt = j * 48 + lane3
            plsc.store_scatter(SFID[b], [tgt], sf)
            plsc.store_scatter(SFID[b], [tgt + 1], sf + 1)
            plsc.store_scatter(SFID[b], [tgt + 2], sf + 2)
            plsc.store_scatter(RFID[b], [tgt], rf)
            plsc.store_scatter(RFID[b], [tgt + 1], rf + 1)
            plsc.store_scatter(RFID[b], [tgt + 2], rf + 2)
            return _
        lax.fori_loop(0, CHUNK // 16, expand16, None)

    def fire_gathers(b):
        for k in range(3 * CHUNK // 128):
            sl = pl.ds(k * 128, 128)
            pltpu.async_copy(
                posf_hbm.at[SFID[b].at[sl]], SCO[b].at[sl], gsem)
            pltpu.async_copy(
                posf_hbm.at[RFID[b].at[sl]], RCO[b].at[sl], gsem)

    def wait_gathers(b):
        for k in range(3 * CHUNK // 128):
            sl = pl.ds(k * 128, 128)
            pltpu.make_async_copy(
                posf_hbm.at[SFID[b].at[sl]], SCO[b].at[sl], gsem).wait()
            pltpu.make_async_copy(
                posf_hbm.at[RFID[b].at[sl]], RCO[b].at[sl], gsem).wait()

    def compute(s, b):
        def edge16(j, _):
            base3 = j * 48
            sx = plsc.load_gather(SCO[b], [base3 + lane3])
            sy = plsc.load_gather(SCO[b], [base3 + lane3 + 1])
            sz = plsc.load_gather(SCO[b], [base3 + lane3 + 2])
            rx = plsc.load_gather(RCO[b], [base3 + lane3])
            ry = plsc.load_gather(RCO[b], [base3 + lane3 + 1])
            rz = plsc.load_gather(RCO[b], [base3 + lane3 + 2])
            dx = rx - sx
            dy = ry - sy
            dz = rz - sz
            r2 = dx * dx + dy * dy + dz * dz
            inv = 1.0 / r2
            e = inv * inv * inv
            en = 2.0 * (e * e - e)
            ridx16 = RIDX[s][pl.ds(j * 16, 16)]
            plsc.addupdate_scatter(accum_v, [ridx16], en)
            return _
        lax.fori_loop(0, CHUNK // 16, edge16, None)

    def step(t, k, start_t3):
        wait_idx((k + 1) % 4)
        expand((k + 1) % 4, (k + 1) % 2)
        fire_gathers((k + 1) % 2)
        wait_gathers(k % 2)
        compute(k % 4, k % 2)
        if start_t3:
            start_idx(t + 3, (k + 3) % 4)

    start_idx(0, 0)
    start_idx(1, 1)
    start_idx(2, 2)
    wait_idx(0)
    expand(0, 0)
    fire_gathers(0)

    def pipe_body(u4, _):
        t0 = u4 * 4
        step(t0 + 0, 0, True)
        step(t0 + 1, 1, True)
        step(t0 + 2, 2, True)
        step(t0 + 3, 3, True)
        return _

    lax.fori_loop(0, (CHUNKS - 4) // 4, pipe_body, None)

    step(CHUNKS - 4, 0, True)
    step(CHUNKS - 3, 1, False)
    step(CHUNKS - 2, 2, False)
    wait_gathers(1)
    compute(3, 1)

    def add_body(i, _):
        plsc.addupdate(accs_v.at[pl.ds(i * 16, 16)], tmp_v[pl.ds(i * 16, 16)])
        return _

    off = sid * MERGE_WIN
    for rr in range(MERGE_ROUNDS):
        pltpu.sync_copy(accum_v.at[pl.ds(rr * MERGE_RN, MERGE_RN)],
                        shared_v.at[pl.ds(sid * MERGE_RN, MERGE_RN)])
        plsc.subcore_barrier()
        pltpu.sync_copy(shared_v.at[pl.ds(off, MERGE_WIN)], accs_v)
        for src in range(1, 16):
            pltpu.sync_copy(
                shared_v.at[pl.ds(src * MERGE_RN + off, MERGE_WIN)], tmp_v)
            lax.fori_loop(0, MERGE_WIN // 16, add_body, None)
        pltpu.sync_copy(accs_v, out_hbm.at[cid, pl.ds(rr * MERGE_RN + off, MERGE_WIN)])
        plsc.subcore_barrier()


def _tc_sum_body(a_ref, o_ref):
    o_ref[...] = a_ref[0] + a_ref[1]


_tc_sum = pl.pallas_call(
    _tc_sum_body,
    out_shape=jax.ShapeDtypeStruct((N_PAD // 128, 128), jnp.float32),
)


def kernel(positions, senders, receivers):
    pos_flat = jnp.concatenate(
        [positions.reshape(-1), jnp.zeros((3 * (N_PAD - N_NODES),), jnp.float32)])
    pad = E_PAD - N_EDGES
    s_pad = jnp.concatenate([senders, jnp.zeros((pad,), jnp.int32)])
    r_pad = jnp.concatenate([receivers, jnp.full((pad,), N_NODES, jnp.int32)])
    partials = _lj_sc(pos_flat, s_pad, r_pad)
    summed = _tc_sum(partials.reshape(2, N_PAD // 128, 128))
    return summed.reshape(-1)[:N_NODES]
